# baseline (speedup 1.0000x reference)
import functools

import jax
import jax.numpy as jnp
from jax import lax
from jax.experimental import pallas as pl
from jax.experimental.pallas import tpu as pltpu
from jax.experimental.pallas import tpu_sc as plsc

NC_ = 2
NS_ = 16
NW_ = NC_ * NS_
L16_ = 16

SCALE_ = 8.0

VOCAB_ = 1000000
D_ = 64
B_ = 4096
L_ = 200

COLS_PER_W_ = 244
FULL_COLS_ = 7812


def _make_transpose_kernel():
    mesh = plsc.VectorSubcoreMesh(core_axis_name="c", subcore_axis_name="s")

    @functools.partial(
        pl.kernel,
        out_type=jax.ShapeDtypeStruct((VOCAB_ * D_,), jnp.float32),
        mesh=mesh,
        scratch_types=[
            pltpu.VMEM((D_, 128), jnp.float32),
            pltpu.VMEM((D_, 128), jnp.float32),
            pltpu.VMEM((128 * D_,), jnp.float32),
            pltpu.VMEM((128 * D_,), jnp.float32),
            pltpu.SemaphoreType.DMA,
            pltpu.SemaphoreType.DMA,
            pltpu.SemaphoreType.DMA,
            pltpu.SemaphoreType.DMA,
        ],
        compiler_params=pltpu.CompilerParams(use_tc_tiling_on_sc=True, needs_layout_passes=False),
    )
    def ka(tbl_t, t_tail, out, tin0, tin1, tout0, tout1, i0, i1, o0, o1):
        wid = lax.axis_index("s") * NC_ + lax.axis_index("c")
        base = wid * COLS_PER_W_
        tins = (tin0, tin1)
        touts = (tout0, tout1)
        isems = (i0, i1)
        osems = (o0, o1)
        iot = lax.iota(jnp.int32, L16_)
        bases = [(iot + b0 * L16_) * D_ for b0 in range(8)]

        def transpose_slab(tin, tout, nb0):
            def cbody(c, carry):
                for b0 in range(nb0):
                    v = tin[c, pl.ds(b0 * L16_, L16_)]
                    plsc.store_scatter(tout, [bases[b0] + c], v)
                return carry
            lax.fori_loop(0, D_, cbody, 0, unroll=4)

        def start_in(k, b):
            j = base + k
            pltpu.async_copy(tbl_t.at[:, pl.ds(j * 128, 128)], tins[b],
                             isems[b])

        def wait_in(b):
            pltpu.make_async_copy(tbl_t.at[:, pl.ds(0, 128)], tins[b],
                                  isems[b]).wait()

        def start_out(k, b):
            j = base + k
            pltpu.async_copy(touts[b], out.at[pl.ds(j * 128 * D_, 128 * D_)],
                             osems[b])

        def drain_out(b):
            pltpu.make_async_copy(touts[b],
                                  out.at[pl.ds(0, 128 * D_)], osems[b]).wait()

        start_in(0, 0)
        start_in(1, 1)
        for b in range(2):
            wait_in(b)
            transpose_slab(tins[b], touts[b], 8)
            start_out(b, b)
            start_in(b + 2, b)

        def body(k2, carry):
            for b in range(2):
                k = 2 * k2 + b
                wait_in(b)
                drain_out(b)
                transpose_slab(tins[b], touts[b], 8)
                start_out(k, b)
                start_in(k + 2, b)
            return carry

        lax.fori_loop(1, COLS_PER_W_ // 2 - 1, body, 0)

        for b in range(2):
            k = COLS_PER_W_ - 2 + b
            wait_in(b)
            drain_out(b)
            transpose_slab(tins[b], touts[b], 8)
            start_out(k, b)
        for b in range(2):
            drain_out(b)

        @pl.when(wid < 4)
        def _():
            j = 32 * COLS_PER_W_ + wid
            pltpu.sync_copy(tbl_t.at[:, pl.ds(j * 128, 128)], tin0)
            transpose_slab(tin0, tout0, 8)
            pltpu.sync_copy(tout0, out.at[pl.ds(j * 128 * D_, 128 * D_)])

        @pl.when(wid == 4)
        def _():
            pltpu.sync_copy(t_tail, tout0.at[pl.ds(0, 64 * D_)])
            pltpu.sync_copy(tout0.at[pl.ds(0, 64 * D_)],
                            out.at[pl.ds(FULL_COLS_ * 128 * D_, 64 * D_)])

    return ka


def _make_gather_kernel():
    mesh = plsc.VectorSubcoreMesh(core_axis_name="c", subcore_axis_name="s")

    @functools.partial(
        pl.kernel,
        out_type=jax.ShapeDtypeStruct((L_, 8, 32, 8, 128), jnp.float32),
        mesh=mesh,
        scratch_types=[
            pltpu.VMEM((128,), jnp.int32),
            pltpu.VMEM((128,), jnp.int32),
            pltpu.VMEM((128, D_), jnp.float32),
            pltpu.VMEM((128, D_), jnp.float32),
            pltpu.VMEM((8, 8, 128), jnp.float32),
            pltpu.VMEM((8, 8, 128), jnp.float32),
            pltpu.SemaphoreType.DMA,
            pltpu.SemaphoreType.DMA,
            pltpu.SemaphoreType.DMA,
            pltpu.SemaphoreType.DMA,
        ],
        compiler_params=pltpu.CompilerParams(use_tc_tiling_on_sc=False, needs_layout_passes=False),
    )
    def kb(t64, tok_lt, out5, x0, x1, g0, g1, s0, s1, gs0, gs1, os0, os1):
        wid = lax.axis_index("s") * NC_ + lax.axis_index("c")
        idxs = (x0, x1)
        gbufs = (g0, g1)
        sbufs = (s0, s1)
        gsems = (gs0, gs1)
        osems = (os0, os1)
        iot = lax.iota(jnp.int32, L16_)
        rowsel = [iot + b0 * L16_ for b0 in range(8)]
        zero16 = iot * 0

        def start(l, b):
            pltpu.sync_copy(tok_lt.at[pl.ds(l * B_ + wid * 128, 128)],
                            idxs[b])
            pltpu.async_copy(t64.at[idxs[b]], gbufs[b], gsems[b])

        def wait_gather(b):
            pltpu.make_async_copy(t64.at[idxs[b]], gbufs[b],
                                  gsems[b]).wait()

        def transpose_scale(g, s):
            def cbody(c, carry):
                cvec = zero16 + c
                c8 = c // 8
                c2 = c - c8 * 8
                for b0 in range(8):
                    v = plsc.load_gather(g, [rowsel[b0], cvec])
                    s[c8, c2, pl.ds(b0 * L16_, L16_)] = v * SCALE_
                return carry
            lax.fori_loop(0, D_, cbody, 0, unroll=4)

        def start_out(l, b):
            pltpu.async_copy(sbufs[b], out5.at[l, :, wid], osems[b])

        def drain_out(b):
            pltpu.make_async_copy(sbufs[b], out5.at[0, :, 0],
                                  osems[b]).wait()

        start(0, 0)
        start(1, 1)
        for b in range(2):
            wait_gather(b)
            transpose_scale(gbufs[b], sbufs[b])
            start_out(b, b)
            start(b + 2, b)

        def body(k2, carry):
            for b in range(2):
                l = 2 * k2 + b
                wait_gather(b)
                drain_out(b)
                transpose_scale(gbufs[b], sbufs[b])
                start_out(l, b)
                start(l + 2, b)
            return carry

        lax.fori_loop(1, L_ // 2 - 1, body, 0)

        for b in range(2):
            l = L_ - 2 + b
            wait_gather(b)
            drain_out(b)
            transpose_scale(gbufs[b], sbufs[b])
            start_out(l, b)
        for b in range(2):
            drain_out(b)

    return kb


@jax.jit
def kernel(tokens, table):
    t_tail = table[FULL_COLS_ * 128:].reshape(64 * D_)
    t_packed = _make_transpose_kernel()(table.T, t_tail)
    t64 = t_packed.reshape(VOCAB_, D_)
    tok_lt = tokens.astype(jnp.int32).T.reshape(B_ * L_)
    out5 = _make_gather_kernel()(t64, tok_lt)
    return out5.transpose(2, 4, 0, 1, 3).reshape(B_, L_, D_)


# speedup vs baseline: 1.6406x; 1.6406x over previous
import functools

import jax
import jax.numpy as jnp
from jax import lax
from jax.experimental import pallas as pl
from jax.experimental.pallas import tpu as pltpu
from jax.experimental.pallas import tpu_sc as plsc

NC_ = 2
NS_ = 16
NW_ = NC_ * NS_
L16_ = 16

SCALE_ = 8.0

VOCAB_ = 1000000
D_ = 64
B_ = 4096
L_ = 200

COLS_PER_W_ = 244
FULL_COLS_ = 7812


def _make_transpose_kernel():
    mesh = plsc.VectorSubcoreMesh(core_axis_name="c", subcore_axis_name="s")

    @functools.partial(
        pl.kernel,
        out_type=jax.ShapeDtypeStruct((VOCAB_ * D_,), jnp.float32),
        mesh=mesh,
        scratch_types=[
            pltpu.VMEM((D_, 128), jnp.float32),
            pltpu.VMEM((D_, 128), jnp.float32),
            pltpu.VMEM((128 * D_,), jnp.float32),
            pltpu.VMEM((128 * D_,), jnp.float32),
            pltpu.SemaphoreType.DMA,
            pltpu.SemaphoreType.DMA,
            pltpu.SemaphoreType.DMA,
            pltpu.SemaphoreType.DMA,
        ],
        compiler_params=pltpu.CompilerParams(use_tc_tiling_on_sc=True, needs_layout_passes=False),
    )
    def ka(tbl_t, t_tail, out, tin0, tin1, tout0, tout1, i0, i1, o0, o1):
        wid = lax.axis_index("s") * NC_ + lax.axis_index("c")
        base = wid * COLS_PER_W_
        tins = (tin0, tin1)
        touts = (tout0, tout1)
        isems = (i0, i1)
        osems = (o0, o1)
        iot = lax.iota(jnp.int32, L16_)
        bases = [(iot + b0 * L16_) * D_ for b0 in range(8)]

        def transpose_slab(tin, tout, nb0):
            @plsc.parallel_loop(0, D_, unroll=4)
            def _(c):
                for b0 in range(nb0):
                    v = tin[c, pl.ds(b0 * L16_, L16_)]
                    plsc.store_scatter(tout, [bases[b0] + c], v)

        def start_in(k, b):
            j = base + k
            pltpu.async_copy(tbl_t.at[:, pl.ds(j * 128, 128)], tins[b],
                             isems[b])

        def wait_in(b):
            pltpu.make_async_copy(tbl_t.at[:, pl.ds(0, 128)], tins[b],
                                  isems[b]).wait()

        def start_out(k, b):
            j = base + k
            pltpu.async_copy(touts[b], out.at[pl.ds(j * 128 * D_, 128 * D_)],
                             osems[b])

        def drain_out(b):
            pltpu.make_async_copy(touts[b],
                                  out.at[pl.ds(0, 128 * D_)], osems[b]).wait()

        start_in(0, 0)
        start_in(1, 1)
        for b in range(2):
            wait_in(b)
            transpose_slab(tins[b], touts[b], 8)
            start_out(b, b)
            start_in(b + 2, b)

        def body(k2, carry):
            for b in range(2):
                k = 2 * k2 + b
                wait_in(b)
                drain_out(b)
                transpose_slab(tins[b], touts[b], 8)
                start_out(k, b)
                start_in(k + 2, b)
            return carry

        lax.fori_loop(1, COLS_PER_W_ // 2 - 1, body, 0)

        for b in range(2):
            k = COLS_PER_W_ - 2 + b
            wait_in(b)
            drain_out(b)
            transpose_slab(tins[b], touts[b], 8)
            start_out(k, b)
        for b in range(2):
            drain_out(b)

        @pl.when(wid < 4)
        def _():
            j = 32 * COLS_PER_W_ + wid
            pltpu.sync_copy(tbl_t.at[:, pl.ds(j * 128, 128)], tin0)
            transpose_slab(tin0, tout0, 8)
            pltpu.sync_copy(tout0, out.at[pl.ds(j * 128 * D_, 128 * D_)])

        @pl.when(wid == 4)
        def _():
            pltpu.sync_copy(t_tail, tout0.at[pl.ds(0, 64 * D_)])
            pltpu.sync_copy(tout0.at[pl.ds(0, 64 * D_)],
                            out.at[pl.ds(FULL_COLS_ * 128 * D_, 64 * D_)])

    return ka


def _make_gather_kernel():
    mesh = plsc.VectorSubcoreMesh(core_axis_name="c", subcore_axis_name="s")

    @functools.partial(
        pl.kernel,
        out_type=jax.ShapeDtypeStruct((L_, 8, 32, 8, 128), jnp.float32),
        mesh=mesh,
        scratch_types=[
            pltpu.VMEM((128,), jnp.int32),
            pltpu.VMEM((128,), jnp.int32),
            pltpu.VMEM((128, D_), jnp.float32),
            pltpu.VMEM((128, D_), jnp.float32),
            pltpu.VMEM((8, 8, 128), jnp.float32),
            pltpu.VMEM((8, 8, 128), jnp.float32),
            pltpu.SemaphoreType.DMA,
            pltpu.SemaphoreType.DMA,
            pltpu.SemaphoreType.DMA,
            pltpu.SemaphoreType.DMA,
        ],
        compiler_params=pltpu.CompilerParams(use_tc_tiling_on_sc=False, needs_layout_passes=False),
    )
    def kb(t64, tok_lt, out5, x0, x1, g0, g1, s0, s1, gs0, gs1, os0, os1):
        wid = lax.axis_index("s") * NC_ + lax.axis_index("c")
        idxs = (x0, x1)
        gbufs = (g0, g1)
        sbufs = (s0, s1)
        gsems = (gs0, gs1)
        osems = (os0, os1)
        iot = lax.iota(jnp.int32, L16_)
        rowsel = [iot + b0 * L16_ for b0 in range(8)]
        zero16 = iot * 0

        def start(l, b):
            pltpu.sync_copy(tok_lt.at[pl.ds(l * B_ + wid * 128, 128)],
                            idxs[b])
            pltpu.async_copy(t64.at[idxs[b]], gbufs[b], gsems[b])

        def wait_gather(b):
            pltpu.make_async_copy(t64.at[idxs[b]], gbufs[b],
                                  gsems[b]).wait()

        def transpose_scale(g, s):
            @plsc.parallel_loop(0, D_, unroll=4)
            def _(c):
                cvec = zero16 + c
                c8 = c // 8
                c2 = c - c8 * 8
                for b0 in range(8):
                    v = plsc.load_gather(g, [rowsel[b0], cvec])
                    s[c8, c2, pl.ds(b0 * L16_, L16_)] = v * SCALE_

        def start_out(l, b):
            pltpu.async_copy(sbufs[b], out5.at[l, :, wid], osems[b])

        def drain_out(b):
            pltpu.make_async_copy(sbufs[b], out5.at[0, :, 0],
                                  osems[b]).wait()

        start(0, 0)
        start(1, 1)
        for b in range(2):
            wait_gather(b)
            transpose_scale(gbufs[b], sbufs[b])
            start_out(b, b)
            start(b + 2, b)

        def body(k2, carry):
            for b in range(2):
                l = 2 * k2 + b
                wait_gather(b)
                drain_out(b)
                transpose_scale(gbufs[b], sbufs[b])
                start_out(l, b)
                start(l + 2, b)
            return carry

        lax.fori_loop(1, L_ // 2 - 1, body, 0)

        for b in range(2):
            l = L_ - 2 + b
            wait_gather(b)
            drain_out(b)
            transpose_scale(gbufs[b], sbufs[b])
            start_out(l, b)
        for b in range(2):
            drain_out(b)

    return kb


@jax.jit
def kernel(tokens, table):
    t_tail = table[FULL_COLS_ * 128:].reshape(64 * D_)
    t_packed = _make_transpose_kernel()(table.T, t_tail)
    t64 = t_packed.reshape(VOCAB_, D_)
    tok_lt = tokens.astype(jnp.int32).T.reshape(B_ * L_)
    out5 = _make_gather_kernel()(t64, tok_lt)
    return out5.transpose(2, 4, 0, 1, 3).reshape(B_, L_, D_)


# speedup vs baseline: 2.5480x; 1.5531x over previous
import functools

import jax
import jax.numpy as jnp
from jax import lax
from jax.experimental import pallas as pl
from jax.experimental.pallas import tpu as pltpu
from jax.experimental.pallas import tpu_sc as plsc

NC_ = 2
NS_ = 16
NW_ = NC_ * NS_
L16_ = 16

SCALE_ = 8.0

VOCAB_ = 1000000
D_ = 64
B_ = 4096
L_ = 200

COLS_PER_W_ = 244
FULL_COLS_ = 7812


def _make_transpose_kernel():
    mesh = plsc.VectorSubcoreMesh(core_axis_name="c", subcore_axis_name="s")

    @functools.partial(
        pl.kernel,
        out_type=jax.ShapeDtypeStruct((VOCAB_ * D_,), jnp.float32),
        mesh=mesh,
        scratch_types=[
            pltpu.VMEM((D_, 129), jnp.float32),
            pltpu.VMEM((D_, 129), jnp.float32),
            pltpu.VMEM((128 * D_,), jnp.float32),
            pltpu.VMEM((128 * D_,), jnp.float32),
            pltpu.SemaphoreType.DMA,
            pltpu.SemaphoreType.DMA,
            pltpu.SemaphoreType.DMA,
            pltpu.SemaphoreType.DMA,
        ],
        compiler_params=pltpu.CompilerParams(use_tc_tiling_on_sc=True, needs_layout_passes=False),
    )
    def ka(tbl_t, t_tail, out, tin0, tin1, tout0, tout1, i0, i1, o0, o1):
        wid = lax.axis_index("s") * NC_ + lax.axis_index("c")
        base = wid * COLS_PER_W_
        tins = (tin0, tin1)
        touts = (tout0, tout1)
        isems = (i0, i1)
        osems = (o0, o1)
        iot = lax.iota(jnp.int32, L16_)
        cvecs = [iot + c0 * L16_ for c0 in range(4)]

        def transpose_slab(tin, tout, ntok):
            @plsc.parallel_loop(0, ntok, unroll=4)
            def _(t):
                tb = t * D_
                tv = iot * 0 + t
                for c0 in range(4):
                    v = plsc.load_gather(tin, [cvecs[c0], tv])
                    tout[pl.ds(tb + c0 * L16_, L16_)] = v

        def start_in(k, b):
            j = base + k
            pltpu.async_copy(tbl_t.at[:, pl.ds(j * 128, 128)],
                             tins[b].at[:, pl.ds(0, 128)], isems[b])

        def wait_in(b):
            pltpu.make_async_copy(tbl_t.at[:, pl.ds(0, 128)],
                                  tins[b].at[:, pl.ds(0, 128)],
                                  isems[b]).wait()

        def start_out(k, b):
            j = base + k
            pltpu.async_copy(touts[b], out.at[pl.ds(j * 128 * D_, 128 * D_)],
                             osems[b])

        def drain_out(b):
            pltpu.make_async_copy(touts[b],
                                  out.at[pl.ds(0, 128 * D_)], osems[b]).wait()

        start_in(0, 0)
        start_in(1, 1)
        for b in range(2):
            wait_in(b)
            transpose_slab(tins[b], touts[b], 128)
            start_out(b, b)
            start_in(b + 2, b)

        def body(k2, carry):
            for b in range(2):
                k = 2 * k2 + b
                wait_in(b)
                drain_out(b)
                transpose_slab(tins[b], touts[b], 128)
                start_out(k, b)
                start_in(k + 2, b)
            return carry

        lax.fori_loop(1, COLS_PER_W_ // 2 - 1, body, 0)

        for b in range(2):
            k = COLS_PER_W_ - 2 + b
            wait_in(b)
            drain_out(b)
            transpose_slab(tins[b], touts[b], 128)
            start_out(k, b)
        for b in range(2):
            drain_out(b)

        @pl.when(wid < 4)
        def _():
            j = 32 * COLS_PER_W_ + wid
            pltpu.sync_copy(tbl_t.at[:, pl.ds(j * 128, 128)],
                            tin0.at[:, pl.ds(0, 128)])
            transpose_slab(tin0, tout0, 128)
            pltpu.sync_copy(tout0, out.at[pl.ds(j * 128 * D_, 128 * D_)])

        @pl.when(wid == 4)
        def _():
            pltpu.sync_copy(t_tail, tout0.at[pl.ds(0, 64 * D_)])
            pltpu.sync_copy(tout0.at[pl.ds(0, 64 * D_)],
                            out.at[pl.ds(FULL_COLS_ * 128 * D_, 64 * D_)])

    return ka


def _make_gather_kernel():
    mesh = plsc.VectorSubcoreMesh(core_axis_name="c", subcore_axis_name="s")

    @functools.partial(
        pl.kernel,
        out_type=jax.ShapeDtypeStruct((L_, 8, 32, 8, 128), jnp.float32),
        mesh=mesh,
        scratch_types=[
            pltpu.VMEM((128,), jnp.int32),
            pltpu.VMEM((128,), jnp.int32),
            pltpu.VMEM((128, D_), jnp.float32),
            pltpu.VMEM((128, D_), jnp.float32),
            pltpu.VMEM((8, 8, 129), jnp.float32),
            pltpu.VMEM((8, 8, 129), jnp.float32),
            pltpu.SemaphoreType.DMA,
            pltpu.SemaphoreType.DMA,
            pltpu.SemaphoreType.DMA,
            pltpu.SemaphoreType.DMA,
        ],
        compiler_params=pltpu.CompilerParams(use_tc_tiling_on_sc=False, needs_layout_passes=False),
    )
    def kb(t64, tok_lt, out5, x0, x1, g0, g1, s0, s1, gs0, gs1, os0, os1):
        wid = lax.axis_index("s") * NC_ + lax.axis_index("c")
        idxs = (x0, x1)
        gbufs = (g0, g1)
        sbufs = (s0, s1)
        gsems = (gs0, gs1)
        osems = (os0, os1)
        iot = lax.iota(jnp.int32, L16_)
        zero16 = iot * 0
        c8vecs = [(iot + c0 * L16_) // 8 for c0 in range(4)]
        c2vecs = [(iot + c0 * L16_) % 8 for c0 in range(4)]

        def start(l, b):
            pltpu.sync_copy(tok_lt.at[pl.ds(l * B_ + wid * 128, 128)],
                            idxs[b])
            pltpu.async_copy(t64.at[idxs[b]], gbufs[b], gsems[b])

        def wait_gather(b):
            pltpu.make_async_copy(t64.at[idxs[b]], gbufs[b],
                                  gsems[b]).wait()

        def transpose_scale(g, s):
            @plsc.parallel_loop(0, 128, unroll=4)
            def _(t):
                tv = zero16 + t
                for c0 in range(4):
                    v = g[t, pl.ds(c0 * L16_, L16_)]
                    plsc.store_scatter(s, [c8vecs[c0], c2vecs[c0], tv],
                                       v * SCALE_)

        def start_out(l, b):
            pltpu.async_copy(sbufs[b].at[:, :, pl.ds(0, 128)],
                             out5.at[l, :, wid], osems[b])

        def drain_out(b):
            pltpu.make_async_copy(sbufs[b].at[:, :, pl.ds(0, 128)],
                                  out5.at[0, :, 0], osems[b]).wait()

        start(0, 0)
        start(1, 1)
        for b in range(2):
            wait_gather(b)
            transpose_scale(gbufs[b], sbufs[b])
            start_out(b, b)
            start(b + 2, b)

        def body(k2, carry):
            for b in range(2):
                l = 2 * k2 + b
                wait_gather(b)
                drain_out(b)
                transpose_scale(gbufs[b], sbufs[b])
                start_out(l, b)
                start(l + 2, b)
            return carry

        lax.fori_loop(1, L_ // 2 - 1, body, 0)

        for b in range(2):
            l = L_ - 2 + b
            wait_gather(b)
            drain_out(b)
            transpose_scale(gbufs[b], sbufs[b])
            start_out(l, b)
        for b in range(2):
            drain_out(b)

    return kb


@jax.jit
def kernel(tokens, table):
    t_tail = table[FULL_COLS_ * 128:].reshape(64 * D_)
    t_packed = _make_transpose_kernel()(table.T, t_tail)
    t64 = t_packed.reshape(VOCAB_, D_)
    tok_lt = tokens.astype(jnp.int32).T.reshape(B_ * L_)
    out5 = _make_gather_kernel()(t64, tok_lt)
    return out5.transpose(2, 4, 0, 1, 3).reshape(B_, L_, D_)


# speedup vs baseline: 3.7740x; 1.4812x over previous
import functools

import jax
import jax.numpy as jnp
from jax import lax
from jax.experimental import pallas as pl
from jax.experimental.pallas import tpu as pltpu
from jax.experimental.pallas import tpu_sc as plsc

NC_ = 2
NS_ = 16
NW_ = NC_ * NS_
L16_ = 16

SCALE_ = 8.0

VOCAB_ = 1000000
D_ = 64
B_ = 4096
L_ = 200

FULL_COLS_ = 7812
COLS_PER_W_ = 244


def _make_restage_kernel():
    mesh = plsc.VectorSubcoreMesh(core_axis_name="c", subcore_axis_name="s")

    @functools.partial(
        pl.kernel,
        out_type=jax.ShapeDtypeStruct((FULL_COLS_, D_, 128), jnp.float32),
        mesh=mesh,
        scratch_types=[
            pltpu.VMEM((D_, 128), jnp.float32),
            pltpu.VMEM((D_, 128), jnp.float32),
            pltpu.VMEM((D_, 128), jnp.float32),
            pltpu.VMEM((D_, 128), jnp.float32),
            pltpu.SemaphoreType.DMA,
            pltpu.SemaphoreType.DMA,
            pltpu.SemaphoreType.DMA,
            pltpu.SemaphoreType.DMA,
            pltpu.SemaphoreType.DMA,
            pltpu.SemaphoreType.DMA,
            pltpu.SemaphoreType.DMA,
            pltpu.SemaphoreType.DMA,
        ],
        compiler_params=pltpu.CompilerParams(use_tc_tiling_on_sc=True,
                                             needs_layout_passes=False),
    )
    def ka1(tbl_t, out3, b0, b1, b2, b3, i0, i1, i2, i3, o0, o1, o2, o3):
        wid = lax.axis_index("s") * NC_ + lax.axis_index("c")
        base = wid * COLS_PER_W_
        bufs = (b0, b1, b2, b3)
        isems = (i0, i1, i2, i3)
        osems = (o0, o1, o2, o3)

        def start_in(k, b):
            pltpu.async_copy(tbl_t.at[:, pl.ds((base + k) * 128, 128)],
                             bufs[b], isems[b])

        def wait_in(b):
            pltpu.make_async_copy(tbl_t.at[:, pl.ds(0, 128)], bufs[b],
                                  isems[b]).wait()

        def start_out(k, b):
            pltpu.async_copy(bufs[b], out3.at[base + k], osems[b])

        def drain_out(b):
            pltpu.make_async_copy(bufs[b], out3.at[0], osems[b]).wait()

        for b in range(4):
            start_in(b, b)
        for b in range(4):
            wait_in(b)
            start_out(b, b)
            start_in(b + 4, b)

        def body(k4, carry):
            for b in range(4):
                k = 4 * k4 + b
                wait_in(b)
                drain_out(b)
                start_out(k, b)
                start_in(k + 4, b)
            return carry

        lax.fori_loop(1, COLS_PER_W_ // 4 - 1, body, 0)

        for b in range(4):
            k = COLS_PER_W_ - 4 + b
            wait_in(b)
            drain_out(b)
            start_out(k, b)
        for b in range(4):
            drain_out(b)

        @pl.when(wid < 4)
        def _():
            j = 32 * COLS_PER_W_ + wid
            pltpu.sync_copy(tbl_t.at[:, pl.ds(j * 128, 128)], b0)
            pltpu.sync_copy(b0, out3.at[j])

    return ka1


def _make_transpose_kernel():
    mesh = plsc.VectorSubcoreMesh(core_axis_name="c", subcore_axis_name="s")

    @functools.partial(
        pl.kernel,
        out_type=jax.ShapeDtypeStruct((VOCAB_, D_), jnp.float32),
        mesh=mesh,
        scratch_types=[
            pltpu.VMEM((D_, 128), jnp.float32),
            pltpu.VMEM((D_, 128), jnp.float32),
            pltpu.VMEM((128, 65), jnp.float32),
            pltpu.VMEM((128, 65), jnp.float32),
            pltpu.VMEM((D_, D_), jnp.float32),
            pltpu.SemaphoreType.DMA,
            pltpu.SemaphoreType.DMA,
            pltpu.SemaphoreType.DMA,
            pltpu.SemaphoreType.DMA,
        ],
        compiler_params=pltpu.CompilerParams(use_tc_tiling_on_sc=False,
                                             needs_layout_passes=False),
    )
    def ka2(slabs, t_tail, out2, tin0, tin1, tout0, tout1, stg,
            i0, i1, o0, o1):
        wid = lax.axis_index("s") * NC_ + lax.axis_index("c")
        base = wid * COLS_PER_W_
        tins = (tin0, tin1)
        touts = (tout0, tout1)
        isems = (i0, i1)
        osems = (o0, o1)
        iot = lax.iota(jnp.int32, L16_)
        tvecs = [iot + t0 * L16_ for t0 in range(8)]

        def transpose_slab(tin, tout):
            @plsc.parallel_loop(0, D_, unroll=4)
            def _(c):
                cv = iot * 0 + c
                for t0 in range(8):
                    v = tin[c, pl.ds(t0 * L16_, L16_)]
                    plsc.store_scatter(tout, [tvecs[t0], cv], v)

        def start_in(k, b):
            pltpu.async_copy(slabs.at[base + k], tins[b], isems[b])

        def wait_in(b):
            pltpu.make_async_copy(slabs.at[0], tins[b], isems[b]).wait()

        def start_out(k, b):
            pltpu.async_copy(touts[b].at[:, pl.ds(0, D_)],
                             out2.at[pl.ds((base + k) * 128, 128)],
                             osems[b])

        def drain_out(b):
            pltpu.make_async_copy(touts[b].at[:, pl.ds(0, D_)],
                                  out2.at[pl.ds(0, 128)], osems[b]).wait()

        start_in(0, 0)
        start_in(1, 1)
        for b in range(2):
            wait_in(b)
            transpose_slab(tins[b], touts[b])
            start_out(b, b)
            start_in(b + 2, b)

        def body(k2, carry):
            for b in range(2):
                k = 2 * k2 + b
                wait_in(b)
                drain_out(b)
                transpose_slab(tins[b], touts[b])
                start_out(k, b)
                start_in(k + 2, b)
            return carry

        lax.fori_loop(1, COLS_PER_W_ // 2 - 1, body, 0)

        for b in range(2):
            k = COLS_PER_W_ - 2 + b
            wait_in(b)
            drain_out(b)
            transpose_slab(tins[b], touts[b])
            start_out(k, b)
        for b in range(2):
            drain_out(b)

        @pl.when(wid < 4)
        def _():
            j = 32 * COLS_PER_W_ + wid
            pltpu.sync_copy(slabs.at[j], tin0)
            transpose_slab(tin0, tout0)
            pltpu.sync_copy(tout0.at[:, pl.ds(0, D_)],
                            out2.at[pl.ds(j * 128, 128)])

        @pl.when(wid == 4)
        def _():
            pltpu.sync_copy(t_tail, stg)
            pltpu.sync_copy(stg, out2.at[pl.ds(FULL_COLS_ * 128, 64)])

    return ka2


def _make_gather_kernel():
    mesh = plsc.VectorSubcoreMesh(core_axis_name="c", subcore_axis_name="s")

    @functools.partial(
        pl.kernel,
        out_type=jax.ShapeDtypeStruct((L_, 8, 32, 8, 128), jnp.float32),
        mesh=mesh,
        scratch_types=[
            pltpu.VMEM((128,), jnp.int32),
            pltpu.VMEM((128,), jnp.int32),
            pltpu.VMEM((128, D_), jnp.float32),
            pltpu.VMEM((128, D_), jnp.float32),
            pltpu.VMEM((8, 8, 129), jnp.float32),
            pltpu.VMEM((8, 8, 129), jnp.float32),
            pltpu.SemaphoreType.DMA,
            pltpu.SemaphoreType.DMA,
            pltpu.SemaphoreType.DMA,
            pltpu.SemaphoreType.DMA,
        ],
        compiler_params=pltpu.CompilerParams(use_tc_tiling_on_sc=False,
                                             needs_layout_passes=False),
    )
    def kb(t64, tok_lt, out5, x0, x1, g0, g1, s0, s1, gs0, gs1, os0, os1):
        wid = lax.axis_index("s") * NC_ + lax.axis_index("c")
        idxs = (x0, x1)
        gbufs = (g0, g1)
        sbufs = (s0, s1)
        gsems = (gs0, gs1)
        osems = (os0, os1)
        iot = lax.iota(jnp.int32, L16_)
        zero16 = iot * 0
        c8vecs = [(iot + c0 * L16_) // 8 for c0 in range(4)]
        c2vecs = [(iot + c0 * L16_) % 8 for c0 in range(4)]

        def start(l, b):
            pltpu.sync_copy(tok_lt.at[pl.ds(l * B_ + wid * 128, 128)],
                            idxs[b])
            pltpu.async_copy(t64.at[idxs[b]], gbufs[b], gsems[b])

        def wait_gather(b):
            pltpu.make_async_copy(t64.at[idxs[b]], gbufs[b],
                                  gsems[b]).wait()

        def transpose_scale(g, s):
            @plsc.parallel_loop(0, 128, unroll=4)
            def _(t):
                tv = zero16 + t
                for c0 in range(4):
                    v = g[t, pl.ds(c0 * L16_, L16_)]
                    plsc.store_scatter(s, [c8vecs[c0], c2vecs[c0], tv],
                                       v * SCALE_)

        def start_out(l, b):
            pltpu.async_copy(sbufs[b].at[:, :, pl.ds(0, 128)],
                             out5.at[l, :, wid], osems[b])

        def drain_out(b):
            pltpu.make_async_copy(sbufs[b].at[:, :, pl.ds(0, 128)],
                                  out5.at[0, :, 0], osems[b]).wait()

        start(0, 0)
        start(1, 1)
        for b in range(2):
            wait_gather(b)
            transpose_scale(gbufs[b], sbufs[b])
            start_out(b, b)
            start(b + 2, b)

        def body(k2, carry):
            for b in range(2):
                l = 2 * k2 + b
                wait_gather(b)
                drain_out(b)
                transpose_scale(gbufs[b], sbufs[b])
                start_out(l, b)
                start(l + 2, b)
            return carry

        lax.fori_loop(1, L_ // 2 - 1, body, 0)

        for b in range(2):
            l = L_ - 2 + b
            wait_gather(b)
            drain_out(b)
            transpose_scale(gbufs[b], sbufs[b])
            start_out(l, b)
        for b in range(2):
            drain_out(b)

    return kb


@jax.jit
def kernel(tokens, table):
    t_tail = table[FULL_COLS_ * 128:]
    slabs = _make_restage_kernel()(table.T)
    t64 = _make_transpose_kernel()(slabs, t_tail)
    tok_lt = tokens.astype(jnp.int32).T.reshape(B_ * L_)
    out5 = _make_gather_kernel()(t64, tok_lt)
    return out5.transpose(2, 4, 0, 1, 3).reshape(B_, L_, D_)
